# Initial kernel scaffold; baseline (speedup 1.0000x reference)
import jax
import jax.numpy as jnp
from jax.experimental import pallas as pl


def kernel(points):
    raise NotImplementedError("write your pallas kernel here")


# speedup vs baseline: 1.9676x; 1.9676x over previous
import functools

import jax
import jax.numpy as jnp
from jax import lax
from jax.experimental import pallas as pl
from jax.experimental.pallas import tpu as pltpu
from jax.experimental.pallas import tpu_sc as plsc


def _scores_kernel(am_ref, bm_ref, nm_ref, ni_ref):
    a = am_ref[0]
    b = bm_ref[0]
    T = b.shape[0]
    RT = a.shape[0]
    s = lax.dot_general(
        a, b, (((1,), (1,)), ((), ())),
        preferred_element_type=jnp.float32)
    nm = jnp.max(s, axis=1)
    jj = lax.broadcasted_iota(jnp.int32, (RT, T), 1)
    ni = jnp.min(jnp.where(s == nm[:, None], jj, T), axis=1)
    nm_ref[0, 0] = nm
    ni_ref[0, 0] = ni.astype(jnp.int32)


def _rank_kernel(nmf_ref, nmt_ref, rank_ref, *, RT):
    nmf = nmf_ref[0, 0]
    nmt = nmt_ref[0, 0]
    T = nmf.shape[0]
    base = pl.program_id(1) * RT
    jj = lax.broadcasted_iota(jnp.int32, (RT, T), 1)
    ii = lax.broadcasted_iota(jnp.int32, (RT, T), 0) + base
    m_i = nmt[:, None]
    m_j = nmf[None, :]
    gt = (m_j > m_i) | ((m_j == m_i) & (jj < ii))
    rank_ref[0, 0] = jnp.sum(gt.astype(jnp.int32), axis=1)


def _plan_kernel(rankf_ref, nif_ref, perm_ref, dstrow_ref, invcnt_ref,
                 *, RT, r):
    rankf = rankf_ref[0, 0]
    nif = nif_ref[0, 0]
    T = rankf.shape[0]
    C = invcnt_ref.shape[2]
    base = pl.program_id(1) * RT
    pp = lax.broadcasted_iota(jnp.int32, (RT, T), 0) + base
    jj = lax.broadcasted_iota(jnp.int32, (RT, T), 1)
    onehot = (rankf[None, :] == pp).astype(jnp.float32)
    perm = jnp.sum(onehot * jj.astype(jnp.float32), axis=1).astype(jnp.int32)
    dstrow = jnp.sum(onehot * nif.astype(jnp.float32)[None, :],
                     axis=1).astype(jnp.int32)
    merged = rankf < r
    hits = ((nif[None, :] == pp) & merged[None, :]).astype(jnp.float32)
    cnt = 1.0 + jnp.sum(hits, axis=1)
    perm_ref[0, 0] = perm + pl.program_id(0) * T
    dstrow_ref[0, 0] = dstrow
    invcnt_ref[0] = jnp.broadcast_to((1.0 / cnt)[:, None], (RT, C))


def _make_merge(B, T, C, r, CP):
    unm_len = T - r
    out_rows = 2 * T - r
    NC, NS = 2, 16
    BPC = B // NC
    MTOK = r // NS
    UTOK = unm_len // NS
    DROW = T // NS
    CW = C // 16

    mesh = plsc.VectorSubcoreMesh(core_axis_name="c", subcore_axis_name="s")

    @functools.partial(
        pl.kernel, mesh=mesh,
        out_type=jax.ShapeDtypeStruct((B * out_rows, C), jnp.float32),
        scratch_types=[
            pltpu.VMEM((MTOK,), jnp.int32),
            pltpu.VMEM((MTOK,), jnp.int32),
            pltpu.VMEM((MTOK, CP), jnp.float32),
            pltpu.VMEM((UTOK,), jnp.int32),
            pltpu.VMEM((UTOK, CP), jnp.float32),
            pltpu.VMEM((DROW, CP), jnp.float32),
            pltpu.VMEM((DROW, C), jnp.float32),
            pltpu.VMEM((DROW, C), jnp.float32),
            pltpu.VMEM_SHARED((T, CP), jnp.float32),
            pltpu.SemaphoreType.DMA,
        ],
    )
    def merge(a2_hbm, b2_hbm, perm_hbm, dstrow_hbm, invcnt_hbm, out_hbm,
              sidx_v, didx_v, srows_v, uidx_v, urows_v, drows_v, nrows_v,
              icnt_v, accum_sh, sem):
        cid = lax.axis_index("c")
        sid = lax.axis_index("s")

        def batch_body(bb, carry):
            b = cid * BPC + bb
            pltpu.sync_copy(b2_hbm.at[pl.ds(b * T + sid * DROW, DROW)],
                            drows_v)
            pltpu.sync_copy(drows_v, accum_sh.at[pl.ds(sid * DROW, DROW)])
            pltpu.sync_copy(
                perm_hbm.at[b, 0, pl.ds(r + sid * UTOK, UTOK)], uidx_v)
            pltpu.async_copy(a2_hbm.at[uidx_v], urows_v, sem).wait()

            def unm_body(i, c2):
                for cw in range(CW):
                    nrows_v[i, pl.ds(cw * 16, 16)] = (
                        urows_v[i, pl.ds(cw * 16, 16)])
                return c2

            lax.fori_loop(0, UTOK, unm_body, 0)
            pltpu.sync_copy(
                nrows_v.at[pl.ds(0, UTOK)],
                out_hbm.at[pl.ds(b * out_rows + sid * UTOK, UTOK)])
            plsc.subcore_barrier()
            pltpu.sync_copy(
                perm_hbm.at[b, 0, pl.ds(sid * MTOK, MTOK)], sidx_v)
            pltpu.sync_copy(
                dstrow_hbm.at[b, 0, pl.ds(sid * MTOK, MTOK)], didx_v)
            pltpu.async_copy(a2_hbm.at[sidx_v], srows_v, sem).wait()
            pltpu.sync_copy(srows_v, accum_sh.at[didx_v], add=True)
            plsc.subcore_barrier()
            pltpu.sync_copy(accum_sh.at[pl.ds(sid * DROW, DROW)], drows_v)
            pltpu.sync_copy(invcnt_hbm.at[pl.ds(b * T + sid * DROW, DROW)],
                            icnt_v)

            def row_body(i, c2):
                for cw in range(CW):
                    nrows_v[i, pl.ds(cw * 16, 16)] = (
                        drows_v[i, pl.ds(cw * 16, 16)]
                        * icnt_v[i, pl.ds(cw * 16, 16)])
                return c2

            lax.fori_loop(0, DROW, row_body, 0)
            pltpu.sync_copy(
                nrows_v,
                out_hbm.at[pl.ds(b * out_rows + unm_len + sid * DROW, DROW)])
            plsc.subcore_barrier()
            return carry

        lax.fori_loop(0, BPC, batch_body, 0)

    return merge


def kernel(points):
    B, N, C = points.shape
    T = N // 2
    r = min(N - 3072, T)
    x = points.reshape(B, T, 2, C)
    a = x[:, :, 0, :]
    b = x[:, :, 1, :]
    metric = points / jnp.linalg.norm(points, axis=-1, keepdims=True)
    xm = metric.reshape(B, T, 2, C)
    am = xm[:, :, 0, :]
    bm = xm[:, :, 1, :]

    RT = 256
    NT = T // RT
    node_max, node_idx = pl.pallas_call(
        _scores_kernel,
        grid=(B, NT),
        in_specs=[
            pl.BlockSpec((1, RT, C), lambda b, t: (b, t, 0)),
            pl.BlockSpec((1, T, C), lambda b, t: (b, 0, 0)),
        ],
        out_specs=[
            pl.BlockSpec((1, 1, RT), lambda b, t: (b, 0, t)),
            pl.BlockSpec((1, 1, RT), lambda b, t: (b, 0, t)),
        ],
        out_shape=[
            jax.ShapeDtypeStruct((B, 1, T), jnp.float32),
            jax.ShapeDtypeStruct((B, 1, T), jnp.int32),
        ],
    )(am, bm)

    rank = pl.pallas_call(
        functools.partial(_rank_kernel, RT=RT),
        grid=(B, NT),
        in_specs=[
            pl.BlockSpec((1, 1, T), lambda b, t: (b, 0, 0)),
            pl.BlockSpec((1, 1, RT), lambda b, t: (b, 0, t)),
        ],
        out_specs=pl.BlockSpec((1, 1, RT), lambda b, t: (b, 0, t)),
        out_shape=jax.ShapeDtypeStruct((B, 1, T), jnp.int32),
    )(node_max, node_max)

    perm, dstrow, invcnt = pl.pallas_call(
        functools.partial(_plan_kernel, RT=RT, r=r),
        grid=(B, NT),
        in_specs=[
            pl.BlockSpec((1, 1, T), lambda b, t: (b, 0, 0)),
            pl.BlockSpec((1, 1, T), lambda b, t: (b, 0, 0)),
        ],
        out_specs=[
            pl.BlockSpec((1, 1, RT), lambda b, t: (b, 0, t)),
            pl.BlockSpec((1, 1, RT), lambda b, t: (b, 0, t)),
            pl.BlockSpec((1, RT, C), lambda b, t: (b, t, 0)),
        ],
        out_shape=[
            jax.ShapeDtypeStruct((B, 1, T), jnp.int32),
            jax.ShapeDtypeStruct((B, 1, T), jnp.int32),
            jax.ShapeDtypeStruct((B, T, C), jnp.float32),
        ],
    )(rank, node_idx)

    CP = 128
    a2p = jnp.pad(a.reshape(B * T, C), ((0, 0), (0, CP - C)))
    b2p = jnp.pad(b.reshape(B * T, C), ((0, 0), (0, CP - C)))
    merge = _make_merge(B, T, C, r, CP)
    out2 = merge(a2p, b2p, perm, dstrow, invcnt.reshape(B * T, C))
    return out2.reshape(B, 2 * T - r, C)


# speedup vs baseline: 2.0871x; 1.0607x over previous
import functools

import jax
import jax.numpy as jnp
from jax import lax
from jax.experimental import pallas as pl
from jax.experimental.pallas import tpu as pltpu
from jax.experimental.pallas import tpu_sc as plsc


def _scores_kernel(am_ref, bm_ref, nm_ref, ni_ref):
    a = am_ref[0]
    b = bm_ref[0]
    T = b.shape[0]
    RT = a.shape[0]
    s = lax.dot_general(
        a, b, (((1,), (1,)), ((), ())),
        preferred_element_type=jnp.float32)
    nm = jnp.max(s, axis=1)
    jj = lax.broadcasted_iota(jnp.int32, (RT, T), 1)
    ni = jnp.min(jnp.where(s == nm[:, None], jj, T), axis=1)
    nm_ref[0, 0] = nm
    ni_ref[0, 0] = ni.astype(jnp.int32)


def _rank_kernel(nmf_ref, nmt_ref, rank_ref, *, RT):
    nmf = nmf_ref[0, 0]
    nmt = nmt_ref[0, 0]
    T = nmf.shape[0]
    base = pl.program_id(1) * RT
    jj = lax.broadcasted_iota(jnp.int32, (RT, T), 1)
    ii = lax.broadcasted_iota(jnp.int32, (RT, T), 0) + base
    m_i = nmt[:, None]
    m_j = nmf[None, :]
    gt = (m_j > m_i) | ((m_j == m_i) & (jj < ii))
    rank_ref[0, 0] = jnp.sum(gt.astype(jnp.int32), axis=1)


def _plan_kernel(rankf_ref, nif_ref, perm_ref, dstrow_ref, *, RT, r):
    rankf = rankf_ref[0, 0]
    nif = nif_ref[0, 0]
    T = rankf.shape[0]
    base = pl.program_id(1) * RT
    pp = lax.broadcasted_iota(jnp.int32, (RT, T), 0) + base
    jj = lax.broadcasted_iota(jnp.int32, (RT, T), 1)
    onehot = (rankf[None, :] == pp).astype(jnp.float32)
    perm = jnp.sum(onehot * jj.astype(jnp.float32), axis=1).astype(jnp.int32)
    dstrow = jnp.sum(onehot * nif.astype(jnp.float32)[None, :],
                     axis=1).astype(jnp.int32)
    perm_ref[0, 0] = perm + pl.program_id(0) * T
    dstrow_ref[0, 0] = dstrow


def _make_merge(B, T, C, r, CP):
    unm_len = T - r
    out_rows = 2 * T - r
    NC, NS = 2, 16
    BPC = B // NC
    MTOK = r // NS
    UTOK = unm_len // NS
    DROW = T // NS
    CW = C // 16

    mesh = plsc.VectorSubcoreMesh(core_axis_name="c", subcore_axis_name="s")

    @functools.partial(
        pl.kernel, mesh=mesh,
        out_type=jax.ShapeDtypeStruct((B * out_rows, C), jnp.float32),
        scratch_types=[
            pltpu.VMEM((MTOK,), jnp.int32),
            pltpu.VMEM((MTOK,), jnp.int32),
            pltpu.VMEM((MTOK, CP), jnp.float32),
            pltpu.VMEM((UTOK,), jnp.int32),
            pltpu.VMEM((UTOK, CP), jnp.float32),
            pltpu.VMEM((DROW, CP), jnp.float32),
            pltpu.VMEM((DROW, C), jnp.float32),
            pltpu.VMEM_SHARED((T, CP), jnp.float32),
            pltpu.SemaphoreType.DMA,
        ],
    )
    def merge(a2_hbm, b2_hbm, perm_hbm, dstrow_hbm, out_hbm,
              sidx_v, didx_v, srows_v, uidx_v, urows_v, drows_v, nrows_v,
              accum_sh, sem):
        cid = lax.axis_index("c")
        sid = lax.axis_index("s")

        def batch_body(bb, carry):
            b = cid * BPC + bb
            pltpu.sync_copy(b2_hbm.at[pl.ds(b * T + sid * DROW, DROW)],
                            drows_v)
            pltpu.sync_copy(drows_v, accum_sh.at[pl.ds(sid * DROW, DROW)])
            pltpu.sync_copy(
                perm_hbm.at[b, 0, pl.ds(r + sid * UTOK, UTOK)], uidx_v)
            pltpu.async_copy(a2_hbm.at[uidx_v], urows_v, sem).wait()

            def unm_body(i, c2):
                for cw in range(CW):
                    nrows_v[i, pl.ds(cw * 16, 16)] = (
                        urows_v[i, pl.ds(cw * 16, 16)])
                return c2

            lax.fori_loop(0, UTOK, unm_body, 0)
            pltpu.sync_copy(
                nrows_v.at[pl.ds(0, UTOK)],
                out_hbm.at[pl.ds(b * out_rows + sid * UTOK, UTOK)])
            plsc.subcore_barrier()
            pltpu.sync_copy(
                perm_hbm.at[b, 0, pl.ds(sid * MTOK, MTOK)], sidx_v)
            pltpu.sync_copy(
                dstrow_hbm.at[b, 0, pl.ds(sid * MTOK, MTOK)], didx_v)
            pltpu.async_copy(a2_hbm.at[sidx_v], srows_v, sem).wait()
            pltpu.sync_copy(srows_v, accum_sh.at[didx_v], add=True)
            plsc.subcore_barrier()
            pltpu.sync_copy(accum_sh.at[pl.ds(sid * DROW, DROW)], drows_v)

            def row_body(i, c2):
                cv = drows_v[i, pl.ds(C, 16)]
                sv = 1.0 / cv
                s = sv[0]
                for cw in range(CW):
                    nrows_v[i, pl.ds(cw * 16, 16)] = (
                        drows_v[i, pl.ds(cw * 16, 16)] * s)
                return c2

            lax.fori_loop(0, DROW, row_body, 0)
            pltpu.sync_copy(
                nrows_v,
                out_hbm.at[pl.ds(b * out_rows + unm_len + sid * DROW, DROW)])
            plsc.subcore_barrier()
            return carry

        lax.fori_loop(0, BPC, batch_body, 0)

    return merge


def kernel(points):
    B, N, C = points.shape
    T = N // 2
    r = min(N - 3072, T)
    x = points.reshape(B, T, 2, C)
    a = x[:, :, 0, :]
    b = x[:, :, 1, :]
    metric = points / jnp.linalg.norm(points, axis=-1, keepdims=True)
    xm = metric.reshape(B, T, 2, C)
    am = xm[:, :, 0, :]
    bm = xm[:, :, 1, :]

    RT = 256
    NT = T // RT
    node_max, node_idx = pl.pallas_call(
        _scores_kernel,
        grid=(B, NT),
        in_specs=[
            pl.BlockSpec((1, RT, C), lambda b, t: (b, t, 0)),
            pl.BlockSpec((1, T, C), lambda b, t: (b, 0, 0)),
        ],
        out_specs=[
            pl.BlockSpec((1, 1, RT), lambda b, t: (b, 0, t)),
            pl.BlockSpec((1, 1, RT), lambda b, t: (b, 0, t)),
        ],
        out_shape=[
            jax.ShapeDtypeStruct((B, 1, T), jnp.float32),
            jax.ShapeDtypeStruct((B, 1, T), jnp.int32),
        ],
    )(am, bm)

    rank = pl.pallas_call(
        functools.partial(_rank_kernel, RT=RT),
        grid=(B, NT),
        in_specs=[
            pl.BlockSpec((1, 1, T), lambda b, t: (b, 0, 0)),
            pl.BlockSpec((1, 1, RT), lambda b, t: (b, 0, t)),
        ],
        out_specs=pl.BlockSpec((1, 1, RT), lambda b, t: (b, 0, t)),
        out_shape=jax.ShapeDtypeStruct((B, 1, T), jnp.int32),
    )(node_max, node_max)

    perm, dstrow = pl.pallas_call(
        functools.partial(_plan_kernel, RT=RT, r=r),
        grid=(B, NT),
        in_specs=[
            pl.BlockSpec((1, 1, T), lambda b, t: (b, 0, 0)),
            pl.BlockSpec((1, 1, T), lambda b, t: (b, 0, 0)),
        ],
        out_specs=[
            pl.BlockSpec((1, 1, RT), lambda b, t: (b, 0, t)),
            pl.BlockSpec((1, 1, RT), lambda b, t: (b, 0, t)),
        ],
        out_shape=[
            jax.ShapeDtypeStruct((B, 1, T), jnp.int32),
            jax.ShapeDtypeStruct((B, 1, T), jnp.int32),
        ],
    )(rank, node_idx)

    CP = 128
    one = jnp.ones((B * T, 1), jnp.float32)
    zpad = jnp.zeros((B * T, CP - C - 1), jnp.float32)
    a2p = jnp.concatenate([a.reshape(B * T, C), one, zpad], axis=1)
    b2p = jnp.concatenate([b.reshape(B * T, C), one, zpad], axis=1)
    merge = _make_merge(B, T, C, r, CP)
    out2 = merge(a2p, b2p, perm, dstrow)
    return out2.reshape(B, 2 * T - r, C)


# speedup vs baseline: 2.2196x; 1.0635x over previous
import functools

import jax
import jax.numpy as jnp
from jax import lax
from jax.experimental import pallas as pl
from jax.experimental.pallas import tpu as pltpu
from jax.experimental.pallas import tpu_sc as plsc


def _scores_kernel(am_ref, bm_ref, nm_ref, ni_ref):
    a = am_ref[0]
    b = bm_ref[0]
    T = b.shape[0]
    RT = a.shape[0]
    s = lax.dot_general(
        a, b, (((1,), (1,)), ((), ())),
        preferred_element_type=jnp.float32)
    nm = jnp.max(s, axis=1)
    jj = lax.broadcasted_iota(jnp.int32, (RT, T), 1)
    ni = jnp.min(jnp.where(s == nm[:, None], jj, T), axis=1)
    nm_ref[0, 0] = nm
    ni_ref[0, 0] = ni.astype(jnp.int32)


def _rank_kernel(nmf_ref, nmt_ref, rank_ref, *, RT):
    nmf = nmf_ref[0, 0]
    nmt = nmt_ref[0, 0]
    T = nmf.shape[0]
    base = pl.program_id(1) * RT
    jj = lax.broadcasted_iota(jnp.int32, (RT, T), 1)
    ii = lax.broadcasted_iota(jnp.int32, (RT, T), 0) + base
    m_i = nmt[:, None]
    m_j = nmf[None, :]
    gt = (m_j > m_i) | ((m_j == m_i) & (jj < ii))
    rank_ref[0, 0] = jnp.sum(gt.astype(jnp.int32), axis=1)


def _plan_kernel(rankf_ref, nif_ref, perm_ref, dstrow_ref, *, RT, r):
    rankf = rankf_ref[0, 0]
    nif = nif_ref[0, 0]
    T = rankf.shape[0]
    base = pl.program_id(1) * RT
    pp = lax.broadcasted_iota(jnp.int32, (RT, T), 0) + base
    jj = lax.broadcasted_iota(jnp.int32, (RT, T), 1)
    onehot = (rankf[None, :] == pp).astype(jnp.float32)
    perm = jnp.sum(onehot * jj.astype(jnp.float32), axis=1).astype(jnp.int32)
    dstrow = jnp.sum(onehot * nif.astype(jnp.float32)[None, :],
                     axis=1).astype(jnp.int32)
    perm_ref[0, 0] = perm + pl.program_id(0) * T
    dstrow_ref[0, 0] = dstrow


def _make_merge(B, T, C, r, CP):
    unm_len = T - r
    out_rows = 2 * T - r
    NC, NS = 2, 16
    BPC = B // NC
    MTOK = r // NS
    UTOK = unm_len // NS
    DROW = T // NS
    CW = C // 16

    mesh = plsc.VectorSubcoreMesh(core_axis_name="c", subcore_axis_name="s")

    @functools.partial(
        pl.kernel, mesh=mesh,
        out_type=jax.ShapeDtypeStruct((B * out_rows, C), jnp.float32),
        scratch_types=[
            pltpu.VMEM((MTOK,), jnp.int32),
            pltpu.VMEM((MTOK,), jnp.int32),
            pltpu.VMEM((MTOK, CP), jnp.float32),
            pltpu.VMEM((UTOK,), jnp.int32),
            pltpu.VMEM((UTOK, CP), jnp.float32),
            pltpu.VMEM((DROW, CP), jnp.float32),
            pltpu.VMEM((DROW, C), jnp.float32),
            pltpu.VMEM_SHARED((T, CP), jnp.float32),
            pltpu.SemaphoreType.DMA,
        ],
    )
    def merge(a2_hbm, b2_hbm, perm_hbm, dstrow_hbm, out_hbm,
              sidx_v, didx_v, srows_v, uidx_v, urows_v, drows_v, nrows_v,
              accum_sh, sem):
        cid = lax.axis_index("c")
        sid = lax.axis_index("s")

        def batch_body(bb, carry):
            b = cid * BPC + bb
            pltpu.sync_copy(b2_hbm.at[pl.ds(b * T + sid * DROW, DROW)],
                            drows_v)
            pltpu.sync_copy(drows_v, accum_sh.at[pl.ds(sid * DROW, DROW)])
            pltpu.sync_copy(
                perm_hbm.at[b, 0, pl.ds(r + sid * UTOK, UTOK)], uidx_v)
            pltpu.async_copy(a2_hbm.at[uidx_v], urows_v, sem).wait()

            def unm_body(i, c2):
                for cw in range(CW):
                    nrows_v[i, pl.ds(cw * 16, 16)] = (
                        urows_v[i, pl.ds(cw * 16, 16)])
                return c2

            lax.fori_loop(0, UTOK, unm_body, 0)
            pltpu.sync_copy(
                nrows_v.at[pl.ds(0, UTOK)],
                out_hbm.at[pl.ds(b * out_rows + sid * UTOK, UTOK)])
            plsc.subcore_barrier()
            pltpu.sync_copy(
                perm_hbm.at[b, 0, pl.ds(sid * MTOK, MTOK)], sidx_v)
            pltpu.sync_copy(
                dstrow_hbm.at[b, 0, pl.ds(sid * MTOK, MTOK)], didx_v)
            pltpu.async_copy(a2_hbm.at[sidx_v], srows_v, sem).wait()
            pltpu.sync_copy(srows_v, accum_sh.at[didx_v], add=True)
            plsc.subcore_barrier()
            pltpu.sync_copy(accum_sh.at[pl.ds(sid * DROW, DROW)], drows_v)

            def row_body(i, c2):
                cv = drows_v[i, pl.ds(C, 16)]
                sv = 1.0 / cv
                s = sv[0]
                for cw in range(CW):
                    nrows_v[i, pl.ds(cw * 16, 16)] = (
                        drows_v[i, pl.ds(cw * 16, 16)] * s)
                return c2

            lax.fori_loop(0, DROW, row_body, 0)
            pltpu.sync_copy(
                nrows_v,
                out_hbm.at[pl.ds(b * out_rows + unm_len + sid * DROW, DROW)])
            plsc.subcore_barrier()
            return carry

        lax.fori_loop(0, BPC, batch_body, 0)

    return merge


def kernel(points):
    B, N, C = points.shape
    T = N // 2
    r = min(N - 3072, T)
    x = points.reshape(B, T, 2, C)
    a = x[:, :, 0, :]
    b = x[:, :, 1, :]
    metric = points / jnp.linalg.norm(points, axis=-1, keepdims=True)
    xm = metric.reshape(B, T, 2, C)
    am = xm[:, :, 0, :]
    bm = xm[:, :, 1, :]

    RT = 512
    NT = T // RT
    node_max, node_idx = pl.pallas_call(
        _scores_kernel,
        grid=(B, NT),
        in_specs=[
            pl.BlockSpec((1, RT, C), lambda b, t: (b, t, 0)),
            pl.BlockSpec((1, T, C), lambda b, t: (b, 0, 0)),
        ],
        out_specs=[
            pl.BlockSpec((1, 1, RT), lambda b, t: (b, 0, t)),
            pl.BlockSpec((1, 1, RT), lambda b, t: (b, 0, t)),
        ],
        out_shape=[
            jax.ShapeDtypeStruct((B, 1, T), jnp.float32),
            jax.ShapeDtypeStruct((B, 1, T), jnp.int32),
        ],
    )(am, bm)

    rank = pl.pallas_call(
        functools.partial(_rank_kernel, RT=RT),
        grid=(B, NT),
        in_specs=[
            pl.BlockSpec((1, 1, T), lambda b, t: (b, 0, 0)),
            pl.BlockSpec((1, 1, RT), lambda b, t: (b, 0, t)),
        ],
        out_specs=pl.BlockSpec((1, 1, RT), lambda b, t: (b, 0, t)),
        out_shape=jax.ShapeDtypeStruct((B, 1, T), jnp.int32),
    )(node_max, node_max)

    perm, dstrow = pl.pallas_call(
        functools.partial(_plan_kernel, RT=RT, r=r),
        grid=(B, NT),
        in_specs=[
            pl.BlockSpec((1, 1, T), lambda b, t: (b, 0, 0)),
            pl.BlockSpec((1, 1, T), lambda b, t: (b, 0, 0)),
        ],
        out_specs=[
            pl.BlockSpec((1, 1, RT), lambda b, t: (b, 0, t)),
            pl.BlockSpec((1, 1, RT), lambda b, t: (b, 0, t)),
        ],
        out_shape=[
            jax.ShapeDtypeStruct((B, 1, T), jnp.int32),
            jax.ShapeDtypeStruct((B, 1, T), jnp.int32),
        ],
    )(rank, node_idx)

    CP = 128
    one = jnp.ones((B * T, 1), jnp.float32)
    zpad = jnp.zeros((B * T, CP - C - 1), jnp.float32)
    a2p = jnp.concatenate([a.reshape(B * T, C), one, zpad], axis=1)
    b2p = jnp.concatenate([b.reshape(B * T, C), one, zpad], axis=1)
    merge = _make_merge(B, T, C, r, CP)
    out2 = merge(a2p, b2p, perm, dstrow)
    return out2.reshape(B, 2 * T - r, C)
